# baseline (speedup 1.0000x reference)
import functools

import jax
import jax.numpy as jnp
from jax import lax
from jax.experimental import pallas as pl
from jax.experimental.pallas import tpu as pltpu
from jax.experimental.pallas import tpu_sc as plsc

BATCH = 16384
D = 32
NC = 2
NS = 16
NW = NC * NS
BPW = BATCH // NW
CHUNK = 128
NCH = BPW // CHUNK


def _sc_body(uids_hbm, iids_hbm, utab_hbm, itab_hbm, out_hbm,
             uid_v, iid_v, urows, irows, out_v, sem):
    wid = lax.axis_index("s") * NC + lax.axis_index("c")
    base = wid * BPW

    pltpu.sync_copy(uids_hbm.at[wid], uid_v)
    pltpu.sync_copy(iids_hbm.at[wid], iid_v)

    copies = []
    for k in range(NCH):
        copies.append(pltpu.async_copy(
            utab_hbm.at[uid_v.at[pl.ds(k * CHUNK, CHUNK)]],
            urows.at[pl.ds(k * CHUNK, CHUNK)], sem))
        copies.append(pltpu.async_copy(
            itab_hbm.at[iid_v.at[pl.ds(k * CHUNK, CHUNK)]],
            irows.at[pl.ds(k * CHUNK, CHUNK)], sem))
    for cp in copies:
        cp.wait()

    lane = lax.iota(jnp.int32, 16)

    def block(blk, carry):
        row = blk * 16 + lane
        acc = jnp.zeros((16,), jnp.float32)
        for j in range(D):
            jj = jnp.full((16,), j, jnp.int32)
            ug = plsc.load_gather(urows, [row, jj])
            ig = plsc.load_gather(irows, [row, jj])
            acc = acc + ug * ig
        out_v[pl.ds(blk * 16, 16)] = acc
        return carry

    lax.fori_loop(0, BPW // 16, block, 0)

    pltpu.sync_copy(out_v, out_hbm.at[pl.ds(base, BPW)])


_sc_call = functools.partial(
    pl.kernel,
    out_type=jax.ShapeDtypeStruct((BATCH,), jnp.float32),
    mesh=plsc.VectorSubcoreMesh(core_axis_name="c", subcore_axis_name="s"),
    compiler_params=pltpu.CompilerParams(
        needs_layout_passes=False, use_tc_tiling_on_sc=False),
    scratch_types=[
        pltpu.VMEM((BPW,), jnp.int32),
        pltpu.VMEM((BPW,), jnp.int32),
        pltpu.VMEM((BPW, D), jnp.float32),
        pltpu.VMEM((BPW, D), jnp.float32),
        pltpu.VMEM((BPW,), jnp.float32),
        pltpu.SemaphoreType.DMA,
    ],
)(_sc_body)


def kernel(user_ids, item_ids, user_table, item_table):
    uids = user_ids.reshape(NW, BPW)
    iids = item_ids.reshape(NW, BPW)
    out = _sc_call(uids, iids, user_table, item_table)
    return out[:, None]
